# baseline (speedup 1.0000x reference)
import functools

import jax
import jax.numpy as jnp
from jax import lax
from jax.experimental import pallas as pl
from jax.experimental.pallas import tpu as pltpu
from jax.experimental.pallas import tpu_sc as plsc

N = 10000
D = 128
E = 320000
NP = 10240
DUMMY = N
NC = 2
NS = 16
NW = NC * NS
CH = 128
TCH = 2560
EP = TCH * CH
K0 = 80
K1 = 80
RPT = NP // NS
ZR = 32
R = 2048
CPW = TCH // NW

_sc_mesh = plsc.VectorSubcoreMesh(
    core_axis_name="c", subcore_axis_name="s", num_cores=NC, num_subcores=NS
)
_sc_params = pltpu.CompilerParams(needs_layout_passes=False)


@functools.partial(
    pl.kernel,
    out_type=jax.ShapeDtypeStruct((NW, NP), jnp.float32),
    mesh=_sc_mesh,
    compiler_params=_sc_params,
    scratch_types=[
        pltpu.VMEM((CPW, CH), jnp.int32),
        pltpu.VMEM((NP,), jnp.float32),
        pltpu.SemaphoreType.DMA,
    ],
)
def _deg_kernel(dst_hbm, out_hbm, dsts, deg, sem):
    c = lax.axis_index("c")
    s = lax.axis_index("s")
    wid = c * NS + s

    idx_cp = pltpu.async_copy(dst_hbm.at[pl.ds(wid * CPW, CPW)], dsts, sem)

    def zero_body(k, carry):
        deg[pl.ds(k * 16, 16)] = jnp.zeros((16,), jnp.float32)
        return carry

    lax.fori_loop(0, NP // 16, zero_body, None)
    idx_cp.wait()

    ones = jnp.full((16,), 1.0, jnp.float32)

    def chunk(j, carry):
        def inner(t, icarry):
            idx = dsts[j, pl.ds(t * 16, 16)]
            plsc.addupdate_scatter(deg, [idx], ones)
            return icarry

        lax.fori_loop(0, CH // 16, inner, None)
        return carry

    lax.fori_loop(0, CPW, chunk, None)
    pltpu.sync_copy(deg, out_hbm.at[wid])


@functools.partial(
    pl.kernel,
    out_type=jax.ShapeDtypeStruct((NC, NP, D), jnp.float32),
    mesh=_sc_mesh,
    compiler_params=_sc_params,
    scratch_types=[
        pltpu.VMEM((CH,), jnp.int32),
        pltpu.VMEM((CH,), jnp.int32),
        pltpu.VMEM((CH, D), jnp.float32),
        pltpu.VMEM((ZR, D), jnp.float32),
        pltpu.VMEM_SHARED((NP, D), jnp.float32),
        pltpu.SemaphoreType.DMA,
    ],
)
def _agg_kernel(y_hbm, src_hbm, dst_hbm, out_hbm, sidx, didx, rows, zbuf, acc,
                sem):
    c = lax.axis_index("c")
    s = lax.axis_index("s")
    r0 = s * RPT

    def zrow(i, carry):
        def zlane(t, icarry):
            zbuf[i, pl.ds(t * 16, 16)] = jnp.zeros((16,), jnp.float32)
            return icarry

        lax.fori_loop(0, D // 16, zlane, None)
        return carry

    lax.fori_loop(0, ZR, zrow, None)

    def zcopy(z, carry):
        pltpu.sync_copy(zbuf, acc.at[pl.ds(r0 + z * ZR, ZR)])
        return carry

    lax.fori_loop(0, RPT // ZR, zcopy, None)
    plsc.subcore_barrier()

    start = (c * NS + s) * CPW

    def chunk(j, carry):
        g = start + j
        pltpu.sync_copy(src_hbm.at[g], sidx)
        pltpu.sync_copy(dst_hbm.at[g], didx)
        pltpu.async_copy(y_hbm.at[sidx], rows, sem).wait()
        pltpu.sync_copy(rows, acc.at[didx], add=True)
        return carry

    lax.fori_loop(0, CPW, chunk, None)
    plsc.subcore_barrier()
    pltpu.sync_copy(acc.at[pl.ds(r0, RPT)], out_hbm.at[c, pl.ds(r0, RPT)])


def _t1_body(x_ref, w_ref, dp_ref, y_ref):
    d = lax.rsqrt(jnp.sum(dp_ref[...], axis=0) + 1.0)
    xw = jnp.dot(x_ref[...], w_ref[...], preferred_element_type=jnp.float32)
    y_ref[...] = xw * d[:, None]


_t1 = pl.pallas_call(
    _t1_body,
    grid=(NP // R,),
    in_specs=[
        pl.BlockSpec((R, D), lambda i: (i, 0)),
        pl.BlockSpec((D, D), lambda i: (0, 0)),
        pl.BlockSpec((NW, R), lambda i: (0, i)),
    ],
    out_specs=pl.BlockSpec((R, D), lambda i: (i, 0)),
    out_shape=jax.ShapeDtypeStruct((NP, D), jnp.float32),
)


def _t2_body(a0_ref, a1_ref, y_ref, dp_ref, b_ref, w_ref, out_ref):
    d = lax.rsqrt(jnp.sum(dp_ref[...], axis=0) + 1.0)
    h = d[:, None] * (a0_ref[...] + a1_ref[...] + y_ref[...]) + b_ref[...]
    h = jnp.maximum(h, 0.0)
    out_ref[...] = (
        jnp.dot(h, w_ref[...], preferred_element_type=jnp.float32) * d[:, None]
    )


_t2 = pl.pallas_call(
    _t2_body,
    grid=(NP // R,),
    in_specs=[
        pl.BlockSpec((R, D), lambda i: (i, 0)),
        pl.BlockSpec((R, D), lambda i: (i, 0)),
        pl.BlockSpec((R, D), lambda i: (i, 0)),
        pl.BlockSpec((NW, R), lambda i: (0, i)),
        pl.BlockSpec((1, D), lambda i: (0, 0)),
        pl.BlockSpec((D, D), lambda i: (0, 0)),
    ],
    out_specs=pl.BlockSpec((R, D), lambda i: (i, 0)),
    out_shape=jax.ShapeDtypeStruct((NP, D), jnp.float32),
)


def _t3_body(a0_ref, a1_ref, y_ref, dp_ref, b_ref, out_ref):
    d = lax.rsqrt(jnp.sum(dp_ref[...], axis=0) + 1.0)
    o = d[:, None] * (a0_ref[...] + a1_ref[...] + y_ref[...]) + b_ref[...]
    out_ref[...] = jnp.maximum(o, 0.0)


_t3 = pl.pallas_call(
    _t3_body,
    grid=(NP // R,),
    in_specs=[
        pl.BlockSpec((R, D), lambda i: (i, 0)),
        pl.BlockSpec((R, D), lambda i: (i, 0)),
        pl.BlockSpec((R, D), lambda i: (i, 0)),
        pl.BlockSpec((NW, R), lambda i: (0, i)),
        pl.BlockSpec((1, D), lambda i: (0, 0)),
    ],
    out_specs=pl.BlockSpec((R, D), lambda i: (i, 0)),
    out_shape=jax.ShapeDtypeStruct((NP, D), jnp.float32),
)


def kernel(x, edge_index, W1, b1, W2, b2):
    src = edge_index[0]
    dst = edge_index[1]
    padn = EP - E
    pad_idx = jnp.full((padn,), DUMMY, jnp.int32)
    src_p = jnp.concatenate([src, pad_idx]).reshape(TCH, CH)
    dst_p = jnp.concatenate([dst, pad_idx]).reshape(TCH, CH)
    x_p = jnp.pad(x, ((0, NP - N), (0, 0)))
    b1r = b1.reshape(1, D)
    b2r = b2.reshape(1, D)

    deg_part = _deg_kernel(dst_p)
    y1 = _t1(x_p, W1, deg_part)
    acc1 = _agg_kernel(y1, src_p, dst_p)
    y2 = _t2(acc1[0], acc1[1], y1, deg_part, b1r, W2)
    acc2 = _agg_kernel(y2, src_p, dst_p)
    out = _t3(acc2[0], acc2[1], y2, deg_part, b2r)
    return out[:N]


# speedup vs baseline: 1.0127x; 1.0127x over previous
import functools

import jax
import jax.numpy as jnp
from jax import lax
from jax.experimental import pallas as pl
from jax.experimental.pallas import tpu as pltpu
from jax.experimental.pallas import tpu_sc as plsc

N = 10000
D = 128
E = 320000
NP = 10240
DUMMY = N
NC = 2
NS = 16
NW = NC * NS
CH = 128
TCH = 2560
EP = TCH * CH
K0 = 80
K1 = 80
RPT = NP // NS
ZR = 32
R = 2048
CPW = TCH // NW
EPW = CPW * CH

_sc_mesh = plsc.VectorSubcoreMesh(
    core_axis_name="c", subcore_axis_name="s", num_cores=NC, num_subcores=NS
)
_sc_params = pltpu.CompilerParams(needs_layout_passes=False)


@functools.partial(
    pl.kernel,
    out_type=jax.ShapeDtypeStruct((NW, NP), jnp.float32),
    mesh=_sc_mesh,
    compiler_params=_sc_params,
    scratch_types=[
        pltpu.VMEM((EPW,), jnp.int32),
        pltpu.VMEM((NP,), jnp.float32),
        pltpu.SemaphoreType.DMA,
    ],
)
def _deg_kernel(dst_hbm, out_hbm, dsts, deg, sem):
    c = lax.axis_index("c")
    s = lax.axis_index("s")
    wid = c * NS + s

    idx_cp = pltpu.async_copy(dst_hbm.at[pl.ds(wid * EPW, EPW)], dsts, sem)

    def zero_body(k, carry):
        deg[pl.ds(k * 16, 16)] = jnp.zeros((16,), jnp.float32)
        return carry

    lax.fori_loop(0, NP // 16, zero_body, None)
    idx_cp.wait()

    ones = jnp.full((16,), 1.0, jnp.float32)

    def chunk(j, carry):
        idx = dsts[pl.ds(j * 16, 16)]
        plsc.addupdate_scatter(deg, [idx], ones)
        return carry

    lax.fori_loop(0, EPW // 16, chunk, None)
    pltpu.sync_copy(deg, out_hbm.at[wid])


@functools.partial(
    pl.kernel,
    out_type=jax.ShapeDtypeStruct((NC, NP, D), jnp.float32),
    mesh=_sc_mesh,
    compiler_params=_sc_params,
    scratch_types=[
        pltpu.VMEM((CH,), jnp.int32),
        pltpu.VMEM((CH,), jnp.int32),
        pltpu.VMEM((CH, D), jnp.float32),
        pltpu.VMEM((ZR, D), jnp.float32),
        pltpu.VMEM_SHARED((NP, D), jnp.float32),
        pltpu.SemaphoreType.DMA,
    ],
)
def _agg_kernel(y_hbm, src_hbm, dst_hbm, out_hbm, sidx, didx, rows, zbuf, acc,
                sem):
    c = lax.axis_index("c")
    s = lax.axis_index("s")
    r0 = s * RPT

    def zrow(i, carry):
        def zlane(t, icarry):
            zbuf[i, pl.ds(t * 16, 16)] = jnp.zeros((16,), jnp.float32)
            return icarry

        lax.fori_loop(0, D // 16, zlane, None)
        return carry

    lax.fori_loop(0, ZR, zrow, None)

    def zcopy(z, carry):
        pltpu.sync_copy(zbuf, acc.at[pl.ds(r0 + z * ZR, ZR)])
        return carry

    lax.fori_loop(0, RPT // ZR, zcopy, None)
    plsc.subcore_barrier()

    base = (c * NS + s) * EPW

    def chunk(j, carry):
        off = base + j * CH
        pltpu.sync_copy(src_hbm.at[pl.ds(off, CH)], sidx)
        pltpu.sync_copy(dst_hbm.at[pl.ds(off, CH)], didx)
        pltpu.async_copy(y_hbm.at[sidx], rows, sem).wait()
        pltpu.sync_copy(rows, acc.at[didx], add=True)
        return carry

    lax.fori_loop(0, CPW, chunk, None)
    plsc.subcore_barrier()
    pltpu.sync_copy(acc.at[pl.ds(r0, RPT)], out_hbm.at[c, pl.ds(r0, RPT)])


def _t1_body(x_ref, w_ref, dp_ref, y_ref):
    d = lax.rsqrt(jnp.sum(dp_ref[...], axis=0) + 1.0)
    xw = jnp.dot(x_ref[...], w_ref[...], preferred_element_type=jnp.float32)
    y_ref[...] = xw * d[:, None]


_t1 = pl.pallas_call(
    _t1_body,
    grid=(NP // R,),
    in_specs=[
        pl.BlockSpec((R, D), lambda i: (i, 0)),
        pl.BlockSpec((D, D), lambda i: (0, 0)),
        pl.BlockSpec((NW, R), lambda i: (0, i)),
    ],
    out_specs=pl.BlockSpec((R, D), lambda i: (i, 0)),
    out_shape=jax.ShapeDtypeStruct((NP, D), jnp.float32),
)


def _t2_body(a0_ref, a1_ref, y_ref, dp_ref, b_ref, w_ref, out_ref):
    d = lax.rsqrt(jnp.sum(dp_ref[...], axis=0) + 1.0)
    h = d[:, None] * (a0_ref[...] + a1_ref[...] + y_ref[...]) + b_ref[...]
    h = jnp.maximum(h, 0.0)
    out_ref[...] = (
        jnp.dot(h, w_ref[...], preferred_element_type=jnp.float32) * d[:, None]
    )


_t2 = pl.pallas_call(
    _t2_body,
    grid=(NP // R,),
    in_specs=[
        pl.BlockSpec((R, D), lambda i: (i, 0)),
        pl.BlockSpec((R, D), lambda i: (i, 0)),
        pl.BlockSpec((R, D), lambda i: (i, 0)),
        pl.BlockSpec((NW, R), lambda i: (0, i)),
        pl.BlockSpec((1, D), lambda i: (0, 0)),
        pl.BlockSpec((D, D), lambda i: (0, 0)),
    ],
    out_specs=pl.BlockSpec((R, D), lambda i: (i, 0)),
    out_shape=jax.ShapeDtypeStruct((NP, D), jnp.float32),
)


def _t3_body(a0_ref, a1_ref, y_ref, dp_ref, b_ref, out_ref):
    d = lax.rsqrt(jnp.sum(dp_ref[...], axis=0) + 1.0)
    o = d[:, None] * (a0_ref[...] + a1_ref[...] + y_ref[...]) + b_ref[...]
    out_ref[...] = jnp.maximum(o, 0.0)


_t3 = pl.pallas_call(
    _t3_body,
    grid=(NP // R,),
    in_specs=[
        pl.BlockSpec((R, D), lambda i: (i, 0)),
        pl.BlockSpec((R, D), lambda i: (i, 0)),
        pl.BlockSpec((R, D), lambda i: (i, 0)),
        pl.BlockSpec((NW, R), lambda i: (0, i)),
        pl.BlockSpec((1, D), lambda i: (0, 0)),
    ],
    out_specs=pl.BlockSpec((R, D), lambda i: (i, 0)),
    out_shape=jax.ShapeDtypeStruct((NP, D), jnp.float32),
)


def kernel(x, edge_index, W1, b1, W2, b2):
    src = edge_index[0]
    dst = edge_index[1]
    padn = EP - E
    pad_idx = jnp.full((padn,), DUMMY, jnp.int32)
    src_p = jnp.concatenate([src, pad_idx])
    dst_p = jnp.concatenate([dst, pad_idx])
    x_p = jnp.pad(x, ((0, NP - N), (0, 0)))
    b1r = b1.reshape(1, D)
    b2r = b2.reshape(1, D)

    deg_part = _deg_kernel(dst_p)
    y1 = _t1(x_p, W1, deg_part)
    acc1 = _agg_kernel(y1, src_p, dst_p)
    y2 = _t2(acc1[0], acc1[1], y1, deg_part, b1r, W2)
    acc2 = _agg_kernel(y2, src_p, dst_p)
    out = _t3(acc2[0], acc2[1], y2, deg_part, b2r)
    return out[:N]


# speedup vs baseline: 1.4001x; 1.3826x over previous
import functools

import jax
import jax.numpy as jnp
from jax import lax
from jax.experimental import pallas as pl
from jax.experimental.pallas import tpu as pltpu
from jax.experimental.pallas import tpu_sc as plsc

N = 10000
D = 128
E = 320000
NP = 10240
DUMMY = N
NC = 2
NS = 16
NW = NC * NS
CH = 128
CPW = 79
EPW = CH * CPW
EP = EPW * NW
RPT = NP // NS
ZR = 64
R = 2048

_sc_mesh = plsc.VectorSubcoreMesh(
    core_axis_name="c", subcore_axis_name="s", num_cores=NC, num_subcores=NS
)
_sc_params = pltpu.CompilerParams(needs_layout_passes=False)


@functools.partial(
    pl.kernel,
    out_type=jax.ShapeDtypeStruct((NW, NP), jnp.float32),
    mesh=_sc_mesh,
    compiler_params=_sc_params,
    scratch_types=[
        pltpu.VMEM((CH,), jnp.int32),
        pltpu.VMEM((NP,), jnp.float32),
    ],
)
def _deg_kernel(dst_hbm, out_hbm, dstv, deg):
    c = lax.axis_index("c")
    s = lax.axis_index("s")
    wid = c * NS + s

    def zero_body(k, carry):
        deg[pl.ds(k * 16, 16)] = jnp.zeros((16,), jnp.float32)
        return carry

    lax.fori_loop(0, NP // 16, zero_body, None)

    ones = jnp.full((16,), 1.0, jnp.float32)

    def chunk(j, carry):
        off = wid * EPW + j * CH
        pltpu.sync_copy(dst_hbm.at[pl.ds(off, CH)], dstv)

        def inner(t, icarry):
            idx = dstv[pl.ds(t * 16, 16)]
            plsc.addupdate_scatter(deg, [idx], ones)
            return icarry

        lax.fori_loop(0, CH // 16, inner, None)
        return carry

    lax.fori_loop(0, CPW, chunk, None)
    pltpu.sync_copy(deg, out_hbm.at[wid])


@functools.partial(
    pl.kernel,
    out_type=jax.ShapeDtypeStruct((NC, NP, D), jnp.float32),
    mesh=_sc_mesh,
    compiler_params=_sc_params,
    scratch_types=[
        pltpu.VMEM((CH,), jnp.int32),
        pltpu.VMEM((CH,), jnp.int32),
        pltpu.VMEM((CH, D), jnp.float32),
        pltpu.VMEM((ZR, D), jnp.float32),
        pltpu.VMEM_SHARED((NP, D), jnp.float32),
        pltpu.SemaphoreType.DMA,
    ],
)
def _agg_kernel(y_hbm, src_hbm, dst_hbm, out_hbm, srcv, dstv, rows, zbuf, acc, sem):
    c = lax.axis_index("c")
    s = lax.axis_index("s")
    wid = c * NS + s

    def zrow(i, carry):
        def zlane(t, icarry):
            zbuf[i, pl.ds(t * 16, 16)] = jnp.zeros((16,), jnp.float32)
            return icarry

        lax.fori_loop(0, D // 16, zlane, None)
        return carry

    lax.fori_loop(0, ZR, zrow, None)

    r0 = s * RPT

    def zcopy(z, carry):
        pltpu.sync_copy(zbuf, acc.at[pl.ds(r0 + z * ZR, ZR)])
        return carry

    lax.fori_loop(0, RPT // ZR, zcopy, None)
    plsc.subcore_barrier()

    def chunk(j, carry):
        off = wid * EPW + j * CH
        pltpu.sync_copy(src_hbm.at[pl.ds(off, CH)], srcv)
        pltpu.sync_copy(dst_hbm.at[pl.ds(off, CH)], dstv)
        pltpu.async_copy(y_hbm.at[srcv], rows, sem).wait()
        pltpu.sync_copy(rows, acc.at[dstv], add=True)
        return carry

    lax.fori_loop(0, CPW, chunk, None)
    plsc.subcore_barrier()
    pltpu.sync_copy(acc.at[pl.ds(r0, RPT)], out_hbm.at[c, pl.ds(r0, RPT)])


def _t1_body(x_ref, w_ref, dp_ref, y_ref):
    d = lax.rsqrt(jnp.sum(dp_ref[...], axis=0) + 1.0)
    xw = jnp.dot(x_ref[...], w_ref[...], preferred_element_type=jnp.float32)
    y_ref[...] = xw * d[:, None]


_t1 = pl.pallas_call(
    _t1_body,
    grid=(NP // R,),
    in_specs=[
        pl.BlockSpec((R, D), lambda i: (i, 0)),
        pl.BlockSpec((D, D), lambda i: (0, 0)),
        pl.BlockSpec((NW, R), lambda i: (0, i)),
    ],
    out_specs=pl.BlockSpec((R, D), lambda i: (i, 0)),
    out_shape=jax.ShapeDtypeStruct((NP, D), jnp.float32),
)


def _t2_body(a0_ref, a1_ref, y_ref, dp_ref, b_ref, w_ref, out_ref):
    d = lax.rsqrt(jnp.sum(dp_ref[...], axis=0) + 1.0)
    h = d[:, None] * (a0_ref[...] + a1_ref[...] + y_ref[...]) + b_ref[...]
    h = jnp.maximum(h, 0.0)
    out_ref[...] = (
        jnp.dot(h, w_ref[...], preferred_element_type=jnp.float32) * d[:, None]
    )


_t2 = pl.pallas_call(
    _t2_body,
    grid=(NP // R,),
    in_specs=[
        pl.BlockSpec((R, D), lambda i: (i, 0)),
        pl.BlockSpec((R, D), lambda i: (i, 0)),
        pl.BlockSpec((R, D), lambda i: (i, 0)),
        pl.BlockSpec((NW, R), lambda i: (0, i)),
        pl.BlockSpec((1, D), lambda i: (0, 0)),
        pl.BlockSpec((D, D), lambda i: (0, 0)),
    ],
    out_specs=pl.BlockSpec((R, D), lambda i: (i, 0)),
    out_shape=jax.ShapeDtypeStruct((NP, D), jnp.float32),
)


def _t3_body(a0_ref, a1_ref, y_ref, dp_ref, b_ref, out_ref):
    d = lax.rsqrt(jnp.sum(dp_ref[...], axis=0) + 1.0)
    o = d[:, None] * (a0_ref[...] + a1_ref[...] + y_ref[...]) + b_ref[...]
    out_ref[...] = jnp.maximum(o, 0.0)


_t3 = pl.pallas_call(
    _t3_body,
    grid=(NP // R,),
    in_specs=[
        pl.BlockSpec((R, D), lambda i: (i, 0)),
        pl.BlockSpec((R, D), lambda i: (i, 0)),
        pl.BlockSpec((R, D), lambda i: (i, 0)),
        pl.BlockSpec((NW, R), lambda i: (0, i)),
        pl.BlockSpec((1, D), lambda i: (0, 0)),
    ],
    out_specs=pl.BlockSpec((R, D), lambda i: (i, 0)),
    out_shape=jax.ShapeDtypeStruct((NP, D), jnp.float32),
)


def kernel(x, edge_index, W1, b1, W2, b2):
    src = edge_index[0]
    dst = edge_index[1]
    padn = EP - E
    pad_idx = jnp.full((padn,), DUMMY, jnp.int32)
    src_p = jnp.concatenate([src, pad_idx])
    dst_p = jnp.concatenate([dst, pad_idx])
    x_p = jnp.pad(x, ((0, NP - N), (0, 0)))
    b1r = b1.reshape(1, D)
    b2r = b2.reshape(1, D)

    deg_part = _deg_kernel(dst_p)
    y1 = _t1(x_p, W1, deg_part)
    acc1 = _agg_kernel(y1, src_p, dst_p)
    y2 = _t2(acc1[0], acc1[1], y1, deg_part, b1r, W2)
    acc2 = _agg_kernel(y2, src_p, dst_p)
    out = _t3(acc2[0], acc2[1], y2, deg_part, b2r)
    return out[:N]
